# baseline (speedup 1.0000x reference)
import math

import jax
import jax.numpy as jnp
from jax import lax
from jax.experimental import pallas as pl

_NBINS = 10
_H = 512
_W = 512
_OUT = 507
_FR = 520
_FC = 640


def _atan2(y, x):
    ax = jnp.abs(x)
    ay = jnp.abs(y)
    mx = jnp.maximum(ax, ay)
    mn = jnp.minimum(ax, ay)
    q = mn / jnp.where(mx == 0.0, 1.0, mx)
    big = q > 0.41421356237309503
    qr = jnp.where(big, (q - 1.0) / (q + 1.0), q)
    z = qr * qr
    poly = ((8.05374449538e-2 * z - 1.38776856032e-1) * z
            + 1.99777106478e-1) * z - 3.33329491539e-1
    a = qr + qr * z * poly + jnp.where(big, 0.7853981633974483, 0.0)
    a = jnp.where(ay > ax, 1.5707963267948966 - a, a)
    a = jnp.where(x < 0.0, math.pi - a, a)
    return jnp.where(y < 0.0, -a, a)


def _hog_body(x_ref, o_ref):
    x = x_ref[0, 0].astype(jnp.bfloat16).astype(jnp.float32)

    xr = jnp.concatenate(
        [jnp.zeros((_H, 1), jnp.float32), x, jnp.zeros((_H, _FC - _W - 1), jnp.float32)],
        axis=1,
    )
    xp = jnp.concatenate(
        [jnp.zeros((1, _FC), jnp.float32), xr, jnp.zeros((_FR - _H - 1, _FC), jnp.float32)],
        axis=0,
    )

    up = jnp.roll(xp, 1, axis=0)
    dn = jnp.roll(xp, -1, axis=0)
    sp = up + 2.0 * xp + dn
    dv = up - dn
    gx = jnp.roll(sp, 1, axis=1) - jnp.roll(sp, -1, axis=1)
    gy = jnp.roll(dv, 1, axis=1) + 2.0 * dv + jnp.roll(dv, -1, axis=1)

    mag = jnp.sqrt(gx * gx + gy * gy)
    ri = lax.broadcasted_iota(jnp.int32, (_FR, _FC), 0)
    ci = lax.broadcasted_iota(jnp.int32, (_FR, _FC), 1)
    valid = (ri >= 1) & (ri <= _H) & (ci >= 1) & (ci <= _W)
    mag = jnp.where(valid, mag, 0.0)

    p = _atan2(gx, gy) * (_NBINS / math.pi)
    fl = jnp.floor(p)
    ce = jnp.ceil(p)
    bq = jnp.mod(fl, 10.0)
    tq = jnp.mod(ce, 10.0)
    f = jnp.mod(p, 10.0)
    b_v = mag * (1.0 - (f - bq))
    t_v = mag * (1.0 - (tq - f))

    b_v = b_v * (1.0 / 64.0)
    t_v = t_v * (1.0 / 64.0)
    for k in range(_NBINS):
        kk = float(k)
        h = jnp.where(bq == kk, b_v, 0.0) + jnp.where(tq == kk, t_v, 0.0)
        s2 = h + jnp.roll(h, -1, axis=1)
        s8 = (s2 + jnp.roll(s2, -2, axis=1)) + (jnp.roll(s2, -4, axis=1) + jnp.roll(s2, -6, axis=1))
        s8 = s8[:, 0:512]
        v2 = s8 + jnp.roll(s8, -1, axis=0)
        v8 = (v2 + jnp.roll(v2, -2, axis=0)) + (jnp.roll(v2, -4, axis=0) + jnp.roll(v2, -6, axis=0))
        o_ref[0, k] = v8[0:_OUT, 0:_OUT]


def kernel(x):
    n = x.shape[0]
    return pl.pallas_call(
        _hog_body,
        grid=(n,),
        in_specs=[pl.BlockSpec((1, 1, _H, _W), lambda i: (i, 0, 0, 0))],
        out_specs=pl.BlockSpec((1, _NBINS, _OUT, _OUT), lambda i: (i, 0, 0, 0)),
        out_shape=jax.ShapeDtypeStruct((n, _NBINS, _OUT, _OUT), jnp.float32),
    )(x)


# speedup vs baseline: 1.2049x; 1.2049x over previous
import math

import jax
import jax.numpy as jnp
from jax import lax
from jax.experimental import pallas as pl

_NBINS = 10
_H = 512
_W = 512
_OUT = 507
_FR = 520
_FC = 640


def _atan2(y, x):
    ax = jnp.abs(x)
    ay = jnp.abs(y)
    mx = jnp.maximum(ax, ay)
    mn = jnp.minimum(ax, ay)
    q = mn / jnp.where(mx == 0.0, 1.0, mx)
    big = q > 0.41421356237309503
    qr = jnp.where(big, (q - 1.0) / (q + 1.0), q)
    z = qr * qr
    poly = ((8.05374449538e-2 * z - 1.38776856032e-1) * z
            + 1.99777106478e-1) * z - 3.33329491539e-1
    a = qr + qr * z * poly + jnp.where(big, 0.7853981633974483, 0.0)
    a = jnp.where(ay > ax, 1.5707963267948966 - a, a)
    a = jnp.where(x < 0.0, math.pi - a, a)
    return jnp.where(y < 0.0, -a, a)


def _hog_body(x_ref, o_ref):
    x = x_ref[0, 0].astype(jnp.bfloat16).astype(jnp.float32)

    xr = jnp.concatenate(
        [jnp.zeros((_H, 1), jnp.float32), x, jnp.zeros((_H, _FC - _W - 1), jnp.float32)],
        axis=1,
    )
    xp = jnp.concatenate(
        [jnp.zeros((1, _FC), jnp.float32), xr, jnp.zeros((_FR - _H - 1, _FC), jnp.float32)],
        axis=0,
    )

    up = jnp.roll(xp, 1, axis=0)
    dn = jnp.roll(xp, -1, axis=0)
    sp = up + 2.0 * xp + dn
    dv = up - dn
    gx = jnp.roll(sp, 1, axis=1) - jnp.roll(sp, -1, axis=1)
    gy = jnp.roll(dv, 1, axis=1) + 2.0 * dv + jnp.roll(dv, -1, axis=1)

    mag = jnp.sqrt(gx * gx + gy * gy)
    ri = lax.broadcasted_iota(jnp.int32, (_FR, _FC), 0)
    ci = lax.broadcasted_iota(jnp.int32, (_FR, _FC), 1)
    valid = (ri >= 1) & (ri <= _H) & (ci >= 1) & (ci <= _W)
    mag = jnp.where(valid, mag, 0.0)

    p = _atan2(gx, gy) * (_NBINS / math.pi)
    fl = jnp.floor(p)
    ce = jnp.ceil(p)
    bq = jnp.mod(fl, 10.0)
    tq = jnp.mod(ce, 10.0)
    f = jnp.mod(p, 10.0)
    b_v = mag * (1.0 - (f - bq))
    t_v = mag * (1.0 - (tq - f))

    b_v = b_v * (1.0 / 64.0)
    t_v = t_v * (1.0 / 64.0)
    for k in range(_NBINS):
        kk = float(k)
        h = jnp.where(bq == kk, b_v, 0.0) + jnp.where(tq == kk, t_v, 0.0)
        h = h + jnp.roll(h, -1, axis=1)
        h = h + jnp.roll(h, -2, axis=1)
        h = h + jnp.roll(h, -4, axis=1)
        h = h[:, 0:512]
        h = h + jnp.roll(h, -1, axis=0)
        h = h + jnp.roll(h, -2, axis=0)
        h = h + jnp.roll(h, -4, axis=0)
        o_ref[0, k] = h[0:_OUT, 0:_OUT]


def kernel(x):
    n = x.shape[0]
    return pl.pallas_call(
        _hog_body,
        grid=(n,),
        in_specs=[pl.BlockSpec((1, 1, _H, _W), lambda i: (i, 0, 0, 0))],
        out_specs=pl.BlockSpec((1, _NBINS, _OUT, _OUT), lambda i: (i, 0, 0, 0)),
        out_shape=jax.ShapeDtypeStruct((n, _NBINS, _OUT, _OUT), jnp.float32),
    )(x)


# speedup vs baseline: 1.5083x; 1.2518x over previous
import math

import jax
import jax.numpy as jnp
from jax import lax
from jax.experimental import pallas as pl

_NBINS = 10
_H = 512
_W = 512
_OUT = 507
_FR = 520
_FC = 640


def _atan2(y, x):
    ax = jnp.abs(x)
    ay = jnp.abs(y)
    mx = jnp.maximum(ax, ay)
    mn = jnp.minimum(ax, ay)
    q = mn / jnp.where(mx == 0.0, 1.0, mx)
    big = q > 0.41421356237309503
    qr = jnp.where(big, (q - 1.0) / (q + 1.0), q)
    z = qr * qr
    poly = ((8.05374449538e-2 * z - 1.38776856032e-1) * z
            + 1.99777106478e-1) * z - 3.33329491539e-1
    a = qr + qr * z * poly + jnp.where(big, 0.7853981633974483, 0.0)
    a = jnp.where(ay > ax, 1.5707963267948966 - a, a)
    a = jnp.where(x < 0.0, math.pi - a, a)
    return jnp.where(y < 0.0, -a, a)


def _hog_body(x_ref, o_ref):
    x = x_ref[0, 0].astype(jnp.bfloat16).astype(jnp.float32)

    xr = jnp.concatenate(
        [jnp.zeros((_H, 1), jnp.float32), x, jnp.zeros((_H, _FC - _W - 1), jnp.float32)],
        axis=1,
    )
    xp = jnp.concatenate(
        [jnp.zeros((1, _FC), jnp.float32), xr, jnp.zeros((_FR - _H - 1, _FC), jnp.float32)],
        axis=0,
    )

    up = jnp.roll(xp, 1, axis=0)
    dn = jnp.roll(xp, -1, axis=0)
    sp = up + 2.0 * xp + dn
    dv = up - dn
    gx = jnp.roll(sp, 1, axis=1) - jnp.roll(sp, -1, axis=1)
    gy = jnp.roll(dv, 1, axis=1) + 2.0 * dv + jnp.roll(dv, -1, axis=1)

    mag = jnp.sqrt(gx * gx + gy * gy)
    ri = lax.broadcasted_iota(jnp.int32, (_FR, _FC), 0)
    ci = lax.broadcasted_iota(jnp.int32, (_FR, _FC), 1)
    valid = (ri >= 1) & (ri <= _H) & (ci >= 1) & (ci <= _W)
    mag = jnp.where(valid, mag, 0.0)

    p = _atan2(gx, gy) * (_NBINS / math.pi)
    fl = jnp.floor(p)
    ce = jnp.ceil(p)
    bq = jnp.mod(fl, 10.0)
    tq = jnp.mod(ce, 10.0)
    f = jnp.mod(p, 10.0)
    b_v = mag * (1.0 - (f - bq))
    t_v = mag * (1.0 - (tq - f))

    bqh = bq.astype(jnp.bfloat16)
    tqh = tq.astype(jnp.bfloat16)
    bvh = (b_v * (1.0 / 64.0)).astype(jnp.bfloat16)
    tvh = (t_v * (1.0 / 64.0)).astype(jnp.bfloat16)
    zero = jnp.zeros_like(bvh)
    for k in range(_NBINS):
        kk = jnp.bfloat16(k)
        h = jnp.where(bqh == kk, bvh, zero) + jnp.where(tqh == kk, tvh, zero)
        h = h + jnp.roll(h, -1, axis=1)
        h = h + jnp.roll(h, -2, axis=1)
        h = h + jnp.roll(h, -4, axis=1)
        h = h[:, 0:512]
        h = h + jnp.roll(h, -1, axis=0)
        h = h + jnp.roll(h, -2, axis=0)
        h = h + jnp.roll(h, -4, axis=0)
        o_ref[0, k] = h[0:_OUT, 0:_OUT].astype(jnp.float32)


def kernel(x):
    n = x.shape[0]
    return pl.pallas_call(
        _hog_body,
        grid=(n,),
        in_specs=[pl.BlockSpec((1, 1, _H, _W), lambda i: (i, 0, 0, 0))],
        out_specs=pl.BlockSpec((1, _NBINS, _OUT, _OUT), lambda i: (i, 0, 0, 0)),
        out_shape=jax.ShapeDtypeStruct((n, _NBINS, _OUT, _OUT), jnp.float32),
    )(x)


# speedup vs baseline: 1.5837x; 1.0500x over previous
import math

import jax
import jax.numpy as jnp
from jax import lax
from jax.experimental import pallas as pl

_NBINS = 10
_H = 512
_W = 512
_OUT = 507
_FR = 520
_FC = 640


def _atan2(y, x):
    ax = jnp.abs(x)
    ay = jnp.abs(y)
    mx = jnp.maximum(ax, ay)
    mn = jnp.minimum(ax, ay)
    q = mn / jnp.where(mx == 0.0, 1.0, mx)
    big = q > 0.41421356237309503
    qr = jnp.where(big, (q - 1.0) / (q + 1.0), q)
    z = qr * qr
    poly = ((8.05374449538e-2 * z - 1.38776856032e-1) * z
            + 1.99777106478e-1) * z - 3.33329491539e-1
    a = qr + qr * z * poly + jnp.where(big, 0.7853981633974483, 0.0)
    a = jnp.where(ay > ax, 1.5707963267948966 - a, a)
    a = jnp.where(x < 0.0, math.pi - a, a)
    return jnp.where(y < 0.0, -a, a)


def _hog_body(x_ref, o_ref):
    x = x_ref[0, 0].astype(jnp.bfloat16).astype(jnp.float32)

    xr = jnp.concatenate(
        [jnp.zeros((_H, 1), jnp.float32), x, jnp.zeros((_H, _FC - _W - 1), jnp.float32)],
        axis=1,
    )
    xp = jnp.concatenate(
        [jnp.zeros((1, _FC), jnp.float32), xr, jnp.zeros((_FR - _H - 1, _FC), jnp.float32)],
        axis=0,
    )

    up = jnp.roll(xp, 1, axis=0)
    dn = jnp.roll(xp, -1, axis=0)
    sp = up + 2.0 * xp + dn
    dv = up - dn
    gx = jnp.roll(sp, 1, axis=1) - jnp.roll(sp, -1, axis=1)
    gy = jnp.roll(dv, 1, axis=1) + 2.0 * dv + jnp.roll(dv, -1, axis=1)

    mag = jnp.sqrt(gx * gx + gy * gy)
    ri = lax.broadcasted_iota(jnp.int32, (_FR, _FC), 0)
    ci = lax.broadcasted_iota(jnp.int32, (_FR, _FC), 1)
    valid = (ri >= 1) & (ri <= _H) & (ci >= 1) & (ci <= _W)
    mag = jnp.where(valid, mag, 0.0)

    p = _atan2(gx, gy) * (_NBINS / math.pi)
    fl = jnp.floor(p)
    ce = jnp.ceil(p)
    def _mod10(v):
        return jnp.where(v >= 10.0, v - 10.0, jnp.where(v < 0.0, v + 10.0, v))

    bq = _mod10(fl)
    tq = _mod10(ce)
    f = _mod10(p)
    b_v = mag * (1.0 - (f - bq))
    t_v = mag * (1.0 - (tq - f))

    bqh = bq.astype(jnp.bfloat16)
    tqh = tq.astype(jnp.bfloat16)
    bvh = (b_v * (1.0 / 64.0)).astype(jnp.bfloat16)
    tvh = (t_v * (1.0 / 64.0)).astype(jnp.bfloat16)
    zero = jnp.zeros_like(bvh)
    for k in range(_NBINS):
        kk = jnp.bfloat16(k)
        h = jnp.where(bqh == kk, bvh, zero) + jnp.where(tqh == kk, tvh, zero)
        h = h + jnp.roll(h, -1, axis=1)
        h = h + jnp.roll(h, -2, axis=1)
        h = h + jnp.roll(h, -4, axis=1)
        h = h[:, 0:512]
        h = h + jnp.roll(h, -1, axis=0)
        h = h + jnp.roll(h, -2, axis=0)
        h = h + jnp.roll(h, -4, axis=0)
        o_ref[0, k] = h[0:_OUT, 0:_OUT].astype(jnp.float32)


def kernel(x):
    n = x.shape[0]
    return pl.pallas_call(
        _hog_body,
        grid=(n,),
        in_specs=[pl.BlockSpec((1, 1, _H, _W), lambda i: (i, 0, 0, 0))],
        out_specs=pl.BlockSpec((1, _NBINS, _OUT, _OUT), lambda i: (i, 0, 0, 0)),
        out_shape=jax.ShapeDtypeStruct((n, _NBINS, _OUT, _OUT), jnp.float32),
    )(x)
